# baseline (speedup 1.0000x reference)
import functools

import jax
import jax.numpy as jnp
from jax import lax
from jax.experimental import pallas as pl
from jax.experimental.pallas import tpu as pltpu
from jax.experimental.pallas import tpu_sc as plsc

LOG2 = 0.6931471805599453

_NC = 2
_NS = 16
_NW = _NC * _NS


def _ssp(v):
    return jax.nn.softplus(v) - LOG2


def _in2f_body(x_ref, w_ref, h_ref):
    h_ref[...] = jnp.dot(x_ref[...], w_ref[...],
                         preferred_element_type=jnp.float32)


def _in2f(x, w, blk):
    n, d = x.shape
    f = w.shape[1]
    return pl.pallas_call(
        _in2f_body,
        grid=(n // blk,),
        in_specs=[
            pl.BlockSpec((blk, d), lambda g: (g, 0)),
            pl.BlockSpec((d, f), lambda g: (0, 0)),
        ],
        out_specs=pl.BlockSpec((blk, f), lambda g: (g, 0)),
        out_shape=jax.ShapeDtypeStruct((n, f), jnp.float32),
    )(x, w)


def _gather_slice(h, idx3, s, e_s, n_chunks, chunk):
    d = h.shape[1]
    epw = e_s // _NW
    mesh = plsc.VectorSubcoreMesh(core_axis_name="c", subcore_axis_name="s")

    unroll = 5
    n_iter = n_chunks // unroll

    @functools.partial(
        pl.kernel,
        out_type=jax.ShapeDtypeStruct((e_s, d), jnp.float32),
        mesh=mesh,
        scratch_types=(
            [pltpu.VMEM((n_chunks, chunk), jnp.int32)]
            + [pltpu.VMEM((chunk, d), jnp.float32) for _ in range(unroll)]
            + [pltpu.SemaphoreType.DMA for _ in range(2 * unroll)]
        ),
    )
    def gather_k(h_hbm, idx_hbm, out_hbm, idx_v, *bufs_sems):
        bufs = bufs_sems[:unroll]
        gsems = bufs_sems[unroll:2 * unroll]
        ssems = bufs_sems[2 * unroll:]
        wid = lax.axis_index("s") * _NC + lax.axis_index("c")
        base = wid * epw
        pltpu.sync_copy(idx_hbm.at[s * _NW + wid], idx_v)

        def s_wait(c, k):
            pltpu.make_async_copy(
                bufs[k], out_hbm.at[pl.ds(base + c * chunk, chunk)],
                ssems[k]).wait()

        def body(i, carry):
            c0 = i * unroll
            for k in range(unroll):
                @pl.when(c0 + k >= unroll)
                def _(k=k, c=c0 + k):
                    s_wait(c - unroll, k)
                pltpu.async_copy(h_hbm.at[idx_v.at[c0 + k]], bufs[k],
                                 gsems[k])
            for k in range(unroll):
                pltpu.make_async_copy(h_hbm.at[idx_v.at[c0 + k]], bufs[k],
                                      gsems[k]).wait()
                pltpu.async_copy(
                    bufs[k],
                    out_hbm.at[pl.ds(base + (c0 + k) * chunk, chunk)],
                    ssems[k])
            return carry

        lax.fori_loop(0, n_iter, body, 0)
        for k in range(unroll):
            s_wait(n_chunks - unroll + k, k)

    return gather_k(h, idx3)


def _conv_body(ws_ref, nw_ref, cin_ref, f_ref, xj_ref, idx_ref, rc_ref,
               w1_ref, b1_ref, w2_ref, b2_ref, conv_ref, *, wwin, blk, g0):
    g = pl.program_id(0)

    @pl.when(g == 0)
    def _():
        conv_ref[...] = cin_ref[...]

    u = _ssp(jnp.dot(f_ref[...], w1_ref[...],
                     preferred_element_type=jnp.float32) + b1_ref[...])
    wij = jnp.dot(u, w2_ref[...],
                  preferred_element_type=jnp.float32) + b2_ref[...]
    x_ij = xj_ref[...] * wij * rc_ref[...]

    idx_row = idx_ref[0]
    ws = ws_ref[g0 + g]
    nw = nw_ref[g0 + g]

    def win_body(w, carry):
        base = ws + w * wwin
        local = idx_row - base
        sel = lax.broadcasted_iota(jnp.int32, (wwin, blk), 0) == local
        st = sel.astype(jnp.float32)
        part = jnp.dot(st, x_ij, preferred_element_type=jnp.float32)
        conv_ref[pl.ds(base, wwin), :] = conv_ref[pl.ds(base, wwin), :] + part
        return carry

    lax.fori_loop(0, nw, win_body, 0)


def _conv_slice(conv_in, f_ij, x_j, idx3, rcut2, ws, nw, w1, b1, w2, b2,
                s, n_blocks, blk, wwin):
    e, r = f_ij.shape
    d = x_j.shape[1]
    f = w1.shape[1]
    n_pad = conv_in.shape[0]
    body = functools.partial(_conv_body, wwin=wwin, blk=blk, g0=s * n_blocks)
    return pl.pallas_call(
        body,
        grid_spec=pltpu.PrefetchScalarGridSpec(
            num_scalar_prefetch=2,
            grid=(n_blocks,),
            in_specs=[
                pl.BlockSpec((n_pad, f), lambda g, *_: (0, 0)),
                pl.BlockSpec((blk, r),
                             lambda g, *_, _s=s, _nb=n_blocks: (_s * _nb + g, 0)),
                pl.BlockSpec((blk, d), lambda g, *_: (g, 0)),
                pl.BlockSpec((1, 1, blk),
                             lambda g, *_, _s=s, _nb=n_blocks: (_s * _nb + g, 0, 0)),
                pl.BlockSpec((blk, 1),
                             lambda g, *_, _s=s, _nb=n_blocks: (_s * _nb + g, 0)),
                pl.BlockSpec((r, f), lambda g, *_: (0, 0)),
                pl.BlockSpec((1, f), lambda g, *_: (0, 0)),
                pl.BlockSpec((f, f), lambda g, *_: (0, 0)),
                pl.BlockSpec((1, f), lambda g, *_: (0, 0)),
            ],
            out_specs=pl.BlockSpec((n_pad, f), lambda g, *_: (0, 0)),
        ),
        out_shape=jax.ShapeDtypeStruct((n_pad, f), jnp.float32),
    )(ws, nw, conv_in, f_ij, x_j, idx3, rcut2, w1, b1, w2, b2)


def _out_body(c_ref, w1_ref, b1_ref, w2_ref, b2_ref, o_ref):
    t = _ssp(jnp.dot(c_ref[...], w1_ref[...],
                     preferred_element_type=jnp.float32) + b1_ref[...])
    o_ref[...] = jnp.dot(t, w2_ref[...],
                         preferred_element_type=jnp.float32) + b2_ref[...]


def _f2out(conv_pad, n, w1, b1, w2, b2, blk):
    f = w1.shape[0]
    d = w1.shape[1]
    return pl.pallas_call(
        _out_body,
        grid=(n // blk,),
        in_specs=[
            pl.BlockSpec((blk, f), lambda g: (g, 0)),
            pl.BlockSpec((f, d), lambda g: (0, 0)),
            pl.BlockSpec((1, d), lambda g: (0, 0)),
            pl.BlockSpec((d, d), lambda g: (0, 0)),
            pl.BlockSpec((1, d), lambda g: (0, 0)),
        ],
        out_specs=pl.BlockSpec((blk, d), lambda g: (g, 0)),
        out_shape=jax.ShapeDtypeStruct((n, d), jnp.float32),
    )(conv_pad, w1, b1, w2, b2)


def kernel(x, f_ij, idx_i, idx_j, rcut_ij, W_in2f, W_f1, b_f1, W_f2, b_f2,
           W_o1, b_o1, W_o2, b_o2):
    n, d = x.shape
    e, r = f_ij.shape
    f = W_in2f.shape[1]

    NSLICE = 1
    BLK = 1280
    WWIN = 128
    CHUNK = 80
    NROW = 1000

    e_s = e // NSLICE
    n_blocks = e_s // BLK
    epw = e_s // _NW
    n_chunks = epw // CHUNK

    idx_i = idx_i.astype(jnp.int32)
    idx_j = idx_j.astype(jnp.int32)
    n_pad = ((n + WWIN - 1) // WWIN) * WWIN

    first = idx_i[0::BLK]
    last = idx_i[BLK - 1::BLK]
    wfirst = first // WWIN
    ws = wfirst * WWIN
    nw = last // WWIN - wfirst + 1

    idx3_i = idx_i.reshape(e // BLK, 1, BLK)
    idx3_j = idx_j.reshape(NSLICE * _NW, n_chunks, CHUNK)
    rcut2 = rcut_ij.reshape(e, 1)
    b_f1r = b_f1.reshape(1, f)
    b_f2r = b_f2.reshape(1, f)
    b_o1r = b_o1.reshape(1, d)
    b_o2r = b_o2.reshape(1, d)

    h = _in2f(x, W_in2f, NROW)
    conv = jnp.zeros((n_pad, f), jnp.float32)
    x_js = [_gather_slice(h, idx3_j, s, e_s, n_chunks, CHUNK)
            for s in range(NSLICE)]
    for s in range(NSLICE):
        conv = _conv_slice(conv, f_ij, x_js[s], idx3_i, rcut2, ws, nw,
                           W_f1, b_f1r, W_f2, b_f2r, s, n_blocks, BLK, WWIN)
    out = _f2out(conv, n, W_o1, b_o1r, W_o2, b_o2r, NROW)
    return out


# speedup vs baseline: 1.0832x; 1.0832x over previous
import functools

import jax
import jax.numpy as jnp
from jax import lax
from jax.experimental import pallas as pl
from jax.experimental.pallas import tpu as pltpu
from jax.experimental.pallas import tpu_sc as plsc

LOG2 = 0.6931471805599453

_NC = 2
_NS = 16
_NW = _NC * _NS


def _ssp(v):
    return jax.nn.softplus(v) - LOG2


def _in2f_body(x_ref, w_ref, h_ref):
    h_ref[...] = jnp.dot(x_ref[...], w_ref[...],
                         preferred_element_type=jnp.float32)


def _in2f(x, w, blk):
    n, d = x.shape
    f = w.shape[1]
    return pl.pallas_call(
        _in2f_body,
        grid=(n // blk,),
        in_specs=[
            pl.BlockSpec((blk, d), lambda g: (g, 0)),
            pl.BlockSpec((d, f), lambda g: (0, 0)),
        ],
        out_specs=pl.BlockSpec((blk, f), lambda g: (g, 0)),
        out_shape=jax.ShapeDtypeStruct((n, f), jnp.float32),
    )(x, w)


def _gather_slice(h, idx3, s, e_s, n_chunks, chunk):
    d = h.shape[1]
    epw = e_s // _NW
    mesh = plsc.VectorSubcoreMesh(core_axis_name="c", subcore_axis_name="s")

    dt = h.dtype

    @functools.partial(
        pl.kernel,
        out_type=jax.ShapeDtypeStruct((e_s, d), dt),
        mesh=mesh,
        scratch_types=[
            pltpu.VMEM((n_chunks, chunk), jnp.int32),
            pltpu.VMEM((chunk, d), dt),
            pltpu.SemaphoreType.DMA,
        ],
    )
    def gather_k(h_hbm, idx_hbm, out_hbm, idx_v, buf_v, gsem):
        wid = lax.axis_index("s") * _NC + lax.axis_index("c")
        base = wid * epw
        pltpu.sync_copy(idx_hbm.at[s * _NW + wid], idx_v)

        def body(c, carry):
            pltpu.async_copy(h_hbm.at[idx_v.at[c]], buf_v, gsem).wait()
            pltpu.sync_copy(buf_v, out_hbm.at[pl.ds(base + c * chunk, chunk)])
            return carry

        lax.fori_loop(0, n_chunks, body, 0)

    return gather_k(h, idx3)


def _conv_body(ws_ref, nw_ref, cin_ref, f_ref, xj_ref, idx_ref, rc_ref,
               w1_ref, b1_ref, w2_ref, b2_ref, conv_ref, *, wwin, blk, g0):
    g = pl.program_id(0)

    @pl.when(g == 0)
    def _():
        conv_ref[...] = cin_ref[...]

    u = _ssp(jnp.dot(f_ref[...], w1_ref[...],
                     preferred_element_type=jnp.float32) + b1_ref[...])
    wij = jnp.dot(u, w2_ref[...],
                  preferred_element_type=jnp.float32) + b2_ref[...]
    x_ij = (xj_ref[...] * wij * rc_ref[...]).astype(jnp.bfloat16)

    idx_row = idx_ref[0]
    ws = ws_ref[g0 + g]
    nw = nw_ref[g0 + g]

    def win_body(w, carry):
        base = ws + w * wwin
        local = idx_row - base
        sel = lax.broadcasted_iota(jnp.int32, (wwin, blk), 0) == local
        st = sel.astype(jnp.bfloat16)
        part = jnp.dot(st, x_ij, preferred_element_type=jnp.float32)
        conv_ref[pl.ds(base, wwin), :] = conv_ref[pl.ds(base, wwin), :] + part
        return carry

    lax.fori_loop(0, nw, win_body, 0)


def _conv_slice(conv_in, f_ij, x_j, idx3, rcut2, ws, nw, w1, b1, w2, b2,
                s, n_blocks, blk, wwin):
    e, r = f_ij.shape
    d = x_j.shape[1]
    f = w1.shape[1]
    n_pad = conv_in.shape[0]
    body = functools.partial(_conv_body, wwin=wwin, blk=blk, g0=s * n_blocks)
    return pl.pallas_call(
        body,
        grid_spec=pltpu.PrefetchScalarGridSpec(
            num_scalar_prefetch=2,
            grid=(n_blocks,),
            in_specs=[
                pl.BlockSpec((n_pad, f), lambda g, *_: (0, 0)),
                pl.BlockSpec((blk, r),
                             lambda g, *_, _s=s, _nb=n_blocks: (_s * _nb + g, 0)),
                pl.BlockSpec((blk, d), lambda g, *_: (g, 0)),
                pl.BlockSpec((1, 1, blk),
                             lambda g, *_, _s=s, _nb=n_blocks: (_s * _nb + g, 0, 0)),
                pl.BlockSpec((blk, 1),
                             lambda g, *_, _s=s, _nb=n_blocks: (_s * _nb + g, 0)),
                pl.BlockSpec((r, f), lambda g, *_: (0, 0)),
                pl.BlockSpec((1, f), lambda g, *_: (0, 0)),
                pl.BlockSpec((f, f), lambda g, *_: (0, 0)),
                pl.BlockSpec((1, f), lambda g, *_: (0, 0)),
            ],
            out_specs=pl.BlockSpec((n_pad, f), lambda g, *_: (0, 0)),
        ),
        out_shape=jax.ShapeDtypeStruct((n_pad, f), jnp.float32),
    )(ws, nw, conv_in, f_ij, x_j, idx3, rcut2, w1, b1, w2, b2)


def _out_body(c_ref, w1_ref, b1_ref, w2_ref, b2_ref, o_ref):
    t = _ssp(jnp.dot(c_ref[...], w1_ref[...],
                     preferred_element_type=jnp.float32) + b1_ref[...])
    o_ref[...] = jnp.dot(t, w2_ref[...],
                         preferred_element_type=jnp.float32) + b2_ref[...]


def _f2out(conv_pad, n, w1, b1, w2, b2, blk):
    f = w1.shape[0]
    d = w1.shape[1]
    return pl.pallas_call(
        _out_body,
        grid=(n // blk,),
        in_specs=[
            pl.BlockSpec((blk, f), lambda g: (g, 0)),
            pl.BlockSpec((f, d), lambda g: (0, 0)),
            pl.BlockSpec((1, d), lambda g: (0, 0)),
            pl.BlockSpec((d, d), lambda g: (0, 0)),
            pl.BlockSpec((1, d), lambda g: (0, 0)),
        ],
        out_specs=pl.BlockSpec((blk, d), lambda g: (g, 0)),
        out_shape=jax.ShapeDtypeStruct((n, d), jnp.float32),
    )(conv_pad, w1, b1, w2, b2)


def kernel(x, f_ij, idx_i, idx_j, rcut_ij, W_in2f, W_f1, b_f1, W_f2, b_f2,
           W_o1, b_o1, W_o2, b_o2):
    n, d = x.shape
    e, r = f_ij.shape
    f = W_in2f.shape[1]

    NSLICE = 1
    BLK = 1280
    WWIN = 128
    CHUNK = 80
    NROW = 1000

    e_s = e // NSLICE
    n_blocks = e_s // BLK
    epw = e_s // _NW
    n_chunks = epw // CHUNK

    idx_i = idx_i.astype(jnp.int32)
    idx_j = idx_j.astype(jnp.int32)
    n_pad = ((n + WWIN - 1) // WWIN) * WWIN

    first = idx_i[0::BLK]
    last = idx_i[BLK - 1::BLK]
    wfirst = first // WWIN
    ws = wfirst * WWIN
    nw = last // WWIN - wfirst + 1

    idx3_i = idx_i.reshape(e // BLK, 1, BLK)
    idx3_j = idx_j.reshape(NSLICE * _NW, n_chunks, CHUNK)
    rcut2 = rcut_ij.reshape(e, 1)
    b_f1r = b_f1.reshape(1, f)
    b_f2r = b_f2.reshape(1, f)
    b_o1r = b_o1.reshape(1, d)
    b_o2r = b_o2.reshape(1, d)

    h = _in2f(x, W_in2f, NROW)
    conv = jnp.zeros((n_pad, f), jnp.float32)
    x_js = [_gather_slice(h, idx3_j, s, e_s, n_chunks, CHUNK)
            for s in range(NSLICE)]
    for s in range(NSLICE):
        conv = _conv_slice(conv, f_ij, x_js[s], idx3_i, rcut2, ws, nw,
                           W_f1, b_f1r, W_f2, b_f2r, s, n_blocks, BLK, WWIN)
    out = _f2out(conv, n, W_o1, b_o1r, W_o2, b_o2r, NROW)
    return out


# speedup vs baseline: 1.2309x; 1.1363x over previous
import functools

import jax
import jax.numpy as jnp
from jax import lax
from jax.experimental import pallas as pl
from jax.experimental.pallas import tpu as pltpu
from jax.experimental.pallas import tpu_sc as plsc

LOG2 = 0.6931471805599453

_NC = 2
_NS = 16
_NW = _NC * _NS


def _ssp(v):
    return jax.nn.softplus(v) - LOG2


def _in2f_body(x_ref, w_ref, h_ref):
    h_ref[...] = jnp.dot(x_ref[...], w_ref[...],
                         preferred_element_type=jnp.float32)


def _in2f(x, w, blk):
    n, d = x.shape
    f = w.shape[1]
    return pl.pallas_call(
        _in2f_body,
        grid=(n // blk,),
        in_specs=[
            pl.BlockSpec((blk, d), lambda g: (g, 0)),
            pl.BlockSpec((d, f), lambda g: (0, 0)),
        ],
        out_specs=pl.BlockSpec((blk, f), lambda g: (g, 0)),
        out_shape=jax.ShapeDtypeStruct((n, f), jnp.float32),
    )(x, w)


def _gather_slice(h, idx3, s, e_s, n_chunks, chunk):
    d = h.shape[1]
    epw = e_s // _NW
    mesh = plsc.VectorSubcoreMesh(core_axis_name="c", subcore_axis_name="s")

    dt = h.dtype

    @functools.partial(
        pl.kernel,
        out_type=jax.ShapeDtypeStruct((e_s, d), dt),
        mesh=mesh,
        scratch_types=[
            pltpu.VMEM((n_chunks, chunk), jnp.int32),
            pltpu.VMEM((chunk, d), dt),
            pltpu.SemaphoreType.DMA,
        ],
    )
    def gather_k(h_hbm, idx_hbm, out_hbm, idx_v, buf_v, gsem):
        wid = lax.axis_index("s") * _NC + lax.axis_index("c")
        base = wid * epw
        pltpu.sync_copy(idx_hbm.at[s * _NW + wid], idx_v)

        def body(c, carry):
            pltpu.async_copy(h_hbm.at[idx_v.at[c]], buf_v, gsem).wait()
            pltpu.sync_copy(buf_v, out_hbm.at[pl.ds(base + c * chunk, chunk)])
            return carry

        lax.fori_loop(0, n_chunks, body, 0)

    return gather_k(h, idx3)


def _conv_body(ws_ref, nw_ref, cin_ref, f_ref, xj_ref, idx_ref, rc_ref,
               w1_ref, b1_ref, w2_ref, b2_ref, conv_ref, *, wwin, blk, g0):
    g = pl.program_id(0)

    @pl.when(g == 0)
    def _():
        conv_ref[...] = cin_ref[...]

    u = _ssp(jnp.dot(f_ref[...], w1_ref[...],
                     preferred_element_type=jnp.float32) + b1_ref[...])
    wij = jnp.dot(u, w2_ref[...],
                  preferred_element_type=jnp.float32) + b2_ref[...]
    x_ij = xj_ref[...] * wij * rc_ref[...]

    idx_row = idx_ref[0]
    ws = ws_ref[g0 + g]
    nw = nw_ref[g0 + g]

    def win_body(w, carry):
        base = ws + w * wwin
        local = idx_row - base
        sel = lax.broadcasted_iota(jnp.int32, (wwin, blk), 0) == local
        st = sel.astype(jnp.float32)
        part = jnp.dot(st, x_ij, preferred_element_type=jnp.float32)
        conv_ref[pl.ds(base, wwin), :] = conv_ref[pl.ds(base, wwin), :] + part
        return carry

    lax.fori_loop(0, nw, win_body, 0)


def _conv_slice(conv_in, f_ij, x_j, idx3, rcut2, ws, nw, w1, b1, w2, b2,
                s, n_blocks, blk, wwin):
    e, r = f_ij.shape
    d = x_j.shape[1]
    f = w1.shape[1]
    n_pad = conv_in.shape[0]
    body = functools.partial(_conv_body, wwin=wwin, blk=blk, g0=s * n_blocks)
    return pl.pallas_call(
        body,
        grid_spec=pltpu.PrefetchScalarGridSpec(
            num_scalar_prefetch=2,
            grid=(n_blocks,),
            in_specs=[
                pl.BlockSpec((n_pad, f), lambda g, *_: (0, 0)),
                pl.BlockSpec((blk, r),
                             lambda g, *_, _s=s, _nb=n_blocks: (_s * _nb + g, 0)),
                pl.BlockSpec((blk, d), lambda g, *_: (g, 0)),
                pl.BlockSpec((1, 1, blk),
                             lambda g, *_, _s=s, _nb=n_blocks: (_s * _nb + g, 0, 0)),
                pl.BlockSpec((blk, 1),
                             lambda g, *_, _s=s, _nb=n_blocks: (_s * _nb + g, 0)),
                pl.BlockSpec((r, f), lambda g, *_: (0, 0)),
                pl.BlockSpec((1, f), lambda g, *_: (0, 0)),
                pl.BlockSpec((f, f), lambda g, *_: (0, 0)),
                pl.BlockSpec((1, f), lambda g, *_: (0, 0)),
            ],
            out_specs=pl.BlockSpec((n_pad, f), lambda g, *_: (0, 0)),
        ),
        out_shape=jax.ShapeDtypeStruct((n_pad, f), jnp.float32),
    )(ws, nw, conv_in, f_ij, x_j, idx3, rcut2, w1, b1, w2, b2)


def _out_body(c_ref, w1_ref, b1_ref, w2_ref, b2_ref, o_ref):
    t = _ssp(jnp.dot(c_ref[...], w1_ref[...],
                     preferred_element_type=jnp.float32) + b1_ref[...])
    o_ref[...] = jnp.dot(t, w2_ref[...],
                         preferred_element_type=jnp.float32) + b2_ref[...]


def _f2out(conv_pad, n, w1, b1, w2, b2, blk):
    f = w1.shape[0]
    d = w1.shape[1]
    return pl.pallas_call(
        _out_body,
        grid=(n // blk,),
        in_specs=[
            pl.BlockSpec((blk, f), lambda g: (g, 0)),
            pl.BlockSpec((f, d), lambda g: (0, 0)),
            pl.BlockSpec((1, d), lambda g: (0, 0)),
            pl.BlockSpec((d, d), lambda g: (0, 0)),
            pl.BlockSpec((1, d), lambda g: (0, 0)),
        ],
        out_specs=pl.BlockSpec((blk, d), lambda g: (g, 0)),
        out_shape=jax.ShapeDtypeStruct((n, d), jnp.float32),
    )(conv_pad, w1, b1, w2, b2)


def kernel(x, f_ij, idx_i, idx_j, rcut_ij, W_in2f, W_f1, b_f1, W_f2, b_f2,
           W_o1, b_o1, W_o2, b_o2):
    n, d = x.shape
    e, r = f_ij.shape
    f = W_in2f.shape[1]

    NSLICE = 1
    BLK = 2560
    WWIN = 128
    CHUNK = 80
    NROW = 1000

    e_s = e // NSLICE
    n_blocks = e_s // BLK
    epw = e_s // _NW
    n_chunks = epw // CHUNK

    idx_i = idx_i.astype(jnp.int32)
    idx_j = idx_j.astype(jnp.int32)
    n_pad = ((n + WWIN - 1) // WWIN) * WWIN

    first = idx_i[0::BLK]
    last = idx_i[BLK - 1::BLK]
    wfirst = first // WWIN
    ws = wfirst * WWIN
    nw = last // WWIN - wfirst + 1

    idx3_i = idx_i.reshape(e // BLK, 1, BLK)
    idx3_j = idx_j.reshape(NSLICE * _NW, n_chunks, CHUNK)
    rcut2 = rcut_ij.reshape(e, 1)
    b_f1r = b_f1.reshape(1, f)
    b_f2r = b_f2.reshape(1, f)
    b_o1r = b_o1.reshape(1, d)
    b_o2r = b_o2.reshape(1, d)

    h = _in2f(x, W_in2f, NROW)
    conv = jnp.zeros((n_pad, f), jnp.float32)
    x_js = [_gather_slice(h, idx3_j, s, e_s, n_chunks, CHUNK)
            for s in range(NSLICE)]
    for s in range(NSLICE):
        conv = _conv_slice(conv, f_ij, x_js[s], idx3_i, rcut2, ws, nw,
                           W_f1, b_f1r, W_f2, b_f2r, s, n_blocks, BLK, WWIN)
    out = _f2out(conv, n, W_o1, b_o1r, W_o2, b_o2r, NROW)
    return out


# speedup vs baseline: 1.2960x; 1.0530x over previous
import functools

import jax
import jax.numpy as jnp
from jax import lax
from jax.experimental import pallas as pl
from jax.experimental.pallas import tpu as pltpu
from jax.experimental.pallas import tpu_sc as plsc

LOG2 = 0.6931471805599453

_NC = 2
_NS = 16
_NW = _NC * _NS


def _ssp(v):
    return jax.nn.softplus(v) - LOG2


def _in2f_body(x_ref, w_ref, h_ref):
    h_ref[...] = jnp.dot(x_ref[...], w_ref[...],
                         preferred_element_type=jnp.float32)


def _in2f(x, w, blk):
    n, d = x.shape
    f = w.shape[1]
    return pl.pallas_call(
        _in2f_body,
        grid=(n // blk,),
        in_specs=[
            pl.BlockSpec((blk, d), lambda g: (g, 0)),
            pl.BlockSpec((d, f), lambda g: (0, 0)),
        ],
        out_specs=pl.BlockSpec((blk, f), lambda g: (g, 0)),
        out_shape=jax.ShapeDtypeStruct((n, f), jnp.float32),
    )(x, w)


def _gather_slice(h, idx3, s, e_s, n_chunks, chunk):
    d = h.shape[1]
    epw = e_s // _NW
    mesh = plsc.VectorSubcoreMesh(core_axis_name="c", subcore_axis_name="s")

    dt = h.dtype

    @functools.partial(
        pl.kernel,
        out_type=jax.ShapeDtypeStruct((e_s, d), dt),
        mesh=mesh,
        scratch_types=[
            pltpu.VMEM((n_chunks, chunk), jnp.int32),
            pltpu.VMEM((chunk, d), dt),
            pltpu.SemaphoreType.DMA,
        ],
    )
    def gather_k(h_hbm, idx_hbm, out_hbm, idx_v, buf_v, gsem):
        wid = lax.axis_index("s") * _NC + lax.axis_index("c")
        base = wid * epw
        pltpu.sync_copy(idx_hbm.at[s * _NW + wid], idx_v)

        def body(c, carry):
            pltpu.async_copy(h_hbm.at[idx_v.at[c]], buf_v, gsem).wait()
            pltpu.sync_copy(buf_v, out_hbm.at[pl.ds(base + c * chunk, chunk)])
            return carry

        lax.fori_loop(0, n_chunks, body, 0)

    return gather_k(h, idx3)


def _conv_body(ws_ref, nw_ref, cin_ref, f_ref, xj_ref, idx_ref, rc_ref,
               w1_ref, b1_ref, w2_ref, b2_ref, conv_ref, *, wwin, blk, g0):
    g = pl.program_id(0)

    @pl.when(g == 0)
    def _():
        conv_ref[...] = cin_ref[...]

    u = _ssp(jnp.dot(f_ref[...], w1_ref[...],
                     preferred_element_type=jnp.float32) + b1_ref[...])
    wij = jnp.dot(u, w2_ref[...],
                  preferred_element_type=jnp.float32) + b2_ref[...]
    x_ij = xj_ref[...] * wij * rc_ref[...]

    idx_row = idx_ref[0]
    ws = ws_ref[g0 + g]
    nw = nw_ref[g0 + g]

    def win_body(w, carry):
        base = ws + w * wwin
        local = idx_row - base
        sel = lax.broadcasted_iota(jnp.int32, (wwin, blk), 0) == local
        st = sel.astype(jnp.float32)
        part = jnp.dot(st, x_ij, preferred_element_type=jnp.float32)
        conv_ref[pl.ds(base, wwin), :] = conv_ref[pl.ds(base, wwin), :] + part
        return carry

    lax.fori_loop(0, nw, win_body, 0)


def _conv_slice(conv_in, f_ij, x_j, idx3, rcut2, ws, nw, w1, b1, w2, b2,
                s, n_blocks, blk, wwin):
    e, r = f_ij.shape
    d = x_j.shape[1]
    f = w1.shape[1]
    n_pad = conv_in.shape[0]
    body = functools.partial(_conv_body, wwin=wwin, blk=blk, g0=s * n_blocks)
    return pl.pallas_call(
        body,
        grid_spec=pltpu.PrefetchScalarGridSpec(
            num_scalar_prefetch=2,
            grid=(n_blocks,),
            in_specs=[
                pl.BlockSpec((n_pad, f), lambda g, *_: (0, 0)),
                pl.BlockSpec((blk, r),
                             lambda g, *_, _s=s, _nb=n_blocks: (_s * _nb + g, 0)),
                pl.BlockSpec((blk, d), lambda g, *_: (g, 0)),
                pl.BlockSpec((1, 1, blk),
                             lambda g, *_, _s=s, _nb=n_blocks: (_s * _nb + g, 0, 0)),
                pl.BlockSpec((blk, 1),
                             lambda g, *_, _s=s, _nb=n_blocks: (_s * _nb + g, 0)),
                pl.BlockSpec((r, f), lambda g, *_: (0, 0)),
                pl.BlockSpec((1, f), lambda g, *_: (0, 0)),
                pl.BlockSpec((f, f), lambda g, *_: (0, 0)),
                pl.BlockSpec((1, f), lambda g, *_: (0, 0)),
            ],
            out_specs=pl.BlockSpec((n_pad, f), lambda g, *_: (0, 0)),
        ),
        out_shape=jax.ShapeDtypeStruct((n_pad, f), jnp.float32),
    )(ws, nw, conv_in, f_ij, x_j, idx3, rcut2, w1, b1, w2, b2)


def _out_body(c_ref, w1_ref, b1_ref, w2_ref, b2_ref, o_ref):
    t = _ssp(jnp.dot(c_ref[...], w1_ref[...],
                     preferred_element_type=jnp.float32) + b1_ref[...])
    o_ref[...] = jnp.dot(t, w2_ref[...],
                         preferred_element_type=jnp.float32) + b2_ref[...]


def _f2out(conv_pad, n, w1, b1, w2, b2, blk):
    f = w1.shape[0]
    d = w1.shape[1]
    return pl.pallas_call(
        _out_body,
        grid=(n // blk,),
        in_specs=[
            pl.BlockSpec((blk, f), lambda g: (g, 0)),
            pl.BlockSpec((f, d), lambda g: (0, 0)),
            pl.BlockSpec((1, d), lambda g: (0, 0)),
            pl.BlockSpec((d, d), lambda g: (0, 0)),
            pl.BlockSpec((1, d), lambda g: (0, 0)),
        ],
        out_specs=pl.BlockSpec((blk, d), lambda g: (g, 0)),
        out_shape=jax.ShapeDtypeStruct((n, d), jnp.float32),
    )(conv_pad, w1, b1, w2, b2)


def kernel(x, f_ij, idx_i, idx_j, rcut_ij, W_in2f, W_f1, b_f1, W_f2, b_f2,
           W_o1, b_o1, W_o2, b_o2):
    n, d = x.shape
    e, r = f_ij.shape
    f = W_in2f.shape[1]

    NSLICE = 1
    BLK = 4000
    WWIN = 128
    CHUNK = 80
    NROW = 1000

    e_s = e // NSLICE
    n_blocks = e_s // BLK
    epw = e_s // _NW
    n_chunks = epw // CHUNK

    idx_i = idx_i.astype(jnp.int32)
    idx_j = idx_j.astype(jnp.int32)
    n_pad = ((n + WWIN - 1) // WWIN) * WWIN

    first = idx_i[0::BLK]
    last = idx_i[BLK - 1::BLK]
    wfirst = first // WWIN
    ws = wfirst * WWIN
    nw = last // WWIN - wfirst + 1

    idx3_i = idx_i.reshape(e // BLK, 1, BLK)
    idx3_j = idx_j.reshape(NSLICE * _NW, n_chunks, CHUNK)
    rcut2 = rcut_ij.reshape(e, 1)
    b_f1r = b_f1.reshape(1, f)
    b_f2r = b_f2.reshape(1, f)
    b_o1r = b_o1.reshape(1, d)
    b_o2r = b_o2.reshape(1, d)

    h = _in2f(x, W_in2f, NROW)
    conv = jnp.zeros((n_pad, f), jnp.float32)
    x_js = [_gather_slice(h, idx3_j, s, e_s, n_chunks, CHUNK)
            for s in range(NSLICE)]
    for s in range(NSLICE):
        conv = _conv_slice(conv, f_ij, x_js[s], idx3_i, rcut2, ws, nw,
                           W_f1, b_f1r, W_f2, b_f2r, s, n_blocks, BLK, WWIN)
    out = _f2out(conv, n, W_o1, b_o1r, W_o2, b_o2r, NROW)
    return out


# speedup vs baseline: 1.3081x; 1.0093x over previous
import functools

import jax
import jax.numpy as jnp
from jax import lax
from jax.experimental import pallas as pl
from jax.experimental.pallas import tpu as pltpu
from jax.experimental.pallas import tpu_sc as plsc

LOG2 = 0.6931471805599453

_NC = 2
_NS = 16
_NW = _NC * _NS


def _ssp(v):
    return jax.nn.softplus(v) - LOG2


def _in2f_body(x_ref, w_ref, h_ref):
    h_ref[...] = jnp.dot(x_ref[...], w_ref[...],
                         preferred_element_type=jnp.float32)


def _in2f(x, w, blk):
    n, d = x.shape
    f = w.shape[1]
    return pl.pallas_call(
        _in2f_body,
        grid=(n // blk,),
        in_specs=[
            pl.BlockSpec((blk, d), lambda g: (g, 0)),
            pl.BlockSpec((d, f), lambda g: (0, 0)),
        ],
        out_specs=pl.BlockSpec((blk, f), lambda g: (g, 0)),
        out_shape=jax.ShapeDtypeStruct((n, f), jnp.float32),
    )(x, w)


def _gather_slice(h, idx3, s, e_s, n_chunks, chunk):
    d = h.shape[1]
    epw = e_s // _NW
    mesh = plsc.VectorSubcoreMesh(core_axis_name="c", subcore_axis_name="s")

    dt = h.dtype

    @functools.partial(
        pl.kernel,
        out_type=jax.ShapeDtypeStruct((e_s, d), dt),
        mesh=mesh,
        scratch_types=[
            pltpu.VMEM((n_chunks, chunk), jnp.int32),
            pltpu.VMEM((chunk, d), dt),
            pltpu.SemaphoreType.DMA,
        ],
    )
    def gather_k(h_hbm, idx_hbm, out_hbm, idx_v, buf_v, gsem):
        wid = lax.axis_index("s") * _NC + lax.axis_index("c")
        base = wid * epw
        pltpu.sync_copy(idx_hbm.at[s * _NW + wid], idx_v)

        def body(c, carry):
            pltpu.async_copy(h_hbm.at[idx_v.at[c]], buf_v, gsem).wait()
            pltpu.sync_copy(buf_v, out_hbm.at[pl.ds(base + c * chunk, chunk)])
            return carry

        lax.fori_loop(0, n_chunks, body, 0)

    return gather_k(h, idx3)


def _conv_body(ws_ref, nw_ref, cin_ref, f_ref, xj_ref, idx_ref, rc_ref,
               w1_ref, b1_ref, w2_ref, b2_ref, conv_ref, *, wwin, blk, g0):
    g = pl.program_id(0)

    @pl.when(g == 0)
    def _():
        conv_ref[...] = cin_ref[...]

    u = _ssp(jnp.dot(f_ref[...], w1_ref[...],
                     preferred_element_type=jnp.float32) + b1_ref[...])
    wij = jnp.dot(u, w2_ref[...],
                  preferred_element_type=jnp.float32) + b2_ref[...]
    x_ij = xj_ref[...] * wij * rc_ref[...]

    idx_row = idx_ref[0]
    ws = ws_ref[g0 + g]
    nw = nw_ref[g0 + g]

    def win_body(w, carry):
        base = ws + w * wwin
        local = idx_row - base
        sel = lax.broadcasted_iota(jnp.int32, (wwin, blk), 0) == local
        st = sel.astype(jnp.float32)
        part = jnp.dot(st, x_ij, preferred_element_type=jnp.float32)
        conv_ref[pl.ds(base, wwin), :] = conv_ref[pl.ds(base, wwin), :] + part
        return carry

    lax.fori_loop(0, nw, win_body, 0)


def _conv_slice(conv_in, f_ij, x_j, idx3, rcut2, ws, nw, w1, b1, w2, b2,
                s, n_blocks, blk, wwin):
    e, r = f_ij.shape
    d = x_j.shape[1]
    f = w1.shape[1]
    n_pad = conv_in.shape[0]
    body = functools.partial(_conv_body, wwin=wwin, blk=blk, g0=s * n_blocks)
    return pl.pallas_call(
        body,
        grid_spec=pltpu.PrefetchScalarGridSpec(
            num_scalar_prefetch=2,
            grid=(n_blocks,),
            in_specs=[
                pl.BlockSpec((n_pad, f), lambda g, *_: (0, 0)),
                pl.BlockSpec((blk, r),
                             lambda g, *_, _s=s, _nb=n_blocks: (_s * _nb + g, 0)),
                pl.BlockSpec((blk, d), lambda g, *_: (g, 0)),
                pl.BlockSpec((1, 1, blk),
                             lambda g, *_, _s=s, _nb=n_blocks: (_s * _nb + g, 0, 0)),
                pl.BlockSpec((blk, 1),
                             lambda g, *_, _s=s, _nb=n_blocks: (_s * _nb + g, 0)),
                pl.BlockSpec((r, f), lambda g, *_: (0, 0)),
                pl.BlockSpec((1, f), lambda g, *_: (0, 0)),
                pl.BlockSpec((f, f), lambda g, *_: (0, 0)),
                pl.BlockSpec((1, f), lambda g, *_: (0, 0)),
            ],
            out_specs=pl.BlockSpec((n_pad, f), lambda g, *_: (0, 0)),
        ),
        out_shape=jax.ShapeDtypeStruct((n_pad, f), jnp.float32),
    )(ws, nw, conv_in, f_ij, x_j, idx3, rcut2, w1, b1, w2, b2)


def _out_body(c_ref, w1_ref, b1_ref, w2_ref, b2_ref, o_ref):
    t = _ssp(jnp.dot(c_ref[...], w1_ref[...],
                     preferred_element_type=jnp.float32) + b1_ref[...])
    o_ref[...] = jnp.dot(t, w2_ref[...],
                         preferred_element_type=jnp.float32) + b2_ref[...]


def _f2out(conv_pad, n, w1, b1, w2, b2, blk):
    f = w1.shape[0]
    d = w1.shape[1]
    return pl.pallas_call(
        _out_body,
        grid=(n // blk,),
        in_specs=[
            pl.BlockSpec((blk, f), lambda g: (g, 0)),
            pl.BlockSpec((f, d), lambda g: (0, 0)),
            pl.BlockSpec((1, d), lambda g: (0, 0)),
            pl.BlockSpec((d, d), lambda g: (0, 0)),
            pl.BlockSpec((1, d), lambda g: (0, 0)),
        ],
        out_specs=pl.BlockSpec((blk, d), lambda g: (g, 0)),
        out_shape=jax.ShapeDtypeStruct((n, d), jnp.float32),
    )(conv_pad, w1, b1, w2, b2)


def kernel(x, f_ij, idx_i, idx_j, rcut_ij, W_in2f, W_f1, b_f1, W_f2, b_f2,
           W_o1, b_o1, W_o2, b_o2):
    n, d = x.shape
    e, r = f_ij.shape
    f = W_in2f.shape[1]

    NSLICE = 1
    BLK = 6400
    WWIN = 128
    CHUNK = 80
    NROW = 1000

    e_s = e // NSLICE
    n_blocks = e_s // BLK
    epw = e_s // _NW
    n_chunks = epw // CHUNK

    idx_i = idx_i.astype(jnp.int32)
    idx_j = idx_j.astype(jnp.int32)
    n_pad = ((n + WWIN - 1) // WWIN) * WWIN

    first = idx_i[0::BLK]
    last = idx_i[BLK - 1::BLK]
    wfirst = first // WWIN
    ws = wfirst * WWIN
    nw = last // WWIN - wfirst + 1

    idx3_i = idx_i.reshape(e // BLK, 1, BLK)
    idx3_j = idx_j.reshape(NSLICE * _NW, n_chunks, CHUNK)
    rcut2 = rcut_ij.reshape(e, 1)
    b_f1r = b_f1.reshape(1, f)
    b_f2r = b_f2.reshape(1, f)
    b_o1r = b_o1.reshape(1, d)
    b_o2r = b_o2.reshape(1, d)

    h = _in2f(x, W_in2f, NROW)
    conv = jnp.zeros((n_pad, f), jnp.float32)
    x_js = [_gather_slice(h, idx3_j, s, e_s, n_chunks, CHUNK)
            for s in range(NSLICE)]
    for s in range(NSLICE):
        conv = _conv_slice(conv, f_ij, x_js[s], idx3_i, rcut2, ws, nw,
                           W_f1, b_f1r, W_f2, b_f2r, s, n_blocks, BLK, WWIN)
    out = _f2out(conv, n, W_o1, b_o1r, W_o2, b_o2r, NROW)
    return out
